# baseline (speedup 1.0000x reference)
import functools

import jax
import jax.numpy as jnp
from jax import lax
from jax.experimental import pallas as pl
from jax.experimental.pallas import tpu as pltpu
from jax.experimental.pallas import tpu_sc as plsc

N = 10000
E = 320000
D = 128
DPE = 4

NTILES = 32
NSUB = 16
NP = 10112
RPT = NP // NSUB
CHW = 128
NCHUNK = 2560
EP = NCHUNK * CHW
CPT = NCHUNK // NTILES
SEG = 16
D16 = 128

_mesh = plsc.VectorSubcoreMesh(core_axis_name="c", subcore_axis_name="s")


@functools.partial(
    pl.kernel,
    mesh=_mesh,
    out_type=jax.ShapeDtypeStruct((2, NP, D), jnp.float32),
    scratch_types=[
        pltpu.VMEM((2, SEG, CHW), jnp.int32),
        pltpu.VMEM((CPT * CHW,), jnp.int32),
        pltpu.VMEM((2 * CHW, D), jnp.float32),
        pltpu.VMEM_SHARED((NP, D), jnp.float32),
        pltpu.SemaphoreType.DMA,
        pltpu.SemaphoreType.DMA,
        pltpu.SemaphoreType.DMA,
        pltpu.SemaphoreType.DMA,
        pltpu.SemaphoreType.DMA,
    ],
)
def _segsum(table_hbm, src_hbm, dst_hbm, zeros_hbm, out_hbm,
            src_v, dst_v, rows_v, acc, sem0, sem1, sem_s0, sem_s1, sem_i):
    cid = lax.axis_index("c")
    sid = lax.axis_index("s")
    wid = sid * 2 + cid
    pltpu.sync_copy(zeros_hbm, acc.at[pl.ds(sid * RPT, RPT)])
    pltpu.sync_copy(dst_hbm.at[wid], dst_v)
    pltpu.sync_copy(src_hbm.at[wid, pl.ds(0, SEG)], src_v.at[0])
    plsc.subcore_barrier()

    sems = (sem0, sem1)
    ssems = (sem_s0, sem_s1)
    for s in range(CPT // SEG):
        pb = s % 2
        if s < CPT // SEG - 1:
            pltpu.async_copy(src_hbm.at[wid, pl.ds((s + 1) * SEG, SEG)],
                             src_v.at[1 - pb], sem_i)
        for b in range(2):
            pltpu.async_copy(table_hbm.at[src_v.at[pb].at[b]],
                             rows_v.at[pl.ds(b * CHW, CHW)], sems[b])

        def pair(p, carry):
            for b in range(2):
                pltpu.make_async_copy(
                    table_hbm.at[src_v.at[pb].at[0]], rows_v.at[pl.ds(b * CHW, CHW)],
                    sems[b]).wait()
            pltpu.sync_copy(
                rows_v,
                acc.at[dst_v.at[pl.ds((s * SEG + 2 * p) * CHW, 2 * CHW)]],
                add=True)
            for b in range(2):
                nj = jnp.minimum(2 * p + b + 2, SEG - 1)
                pltpu.async_copy(table_hbm.at[src_v.at[pb].at[nj]],
                                 rows_v.at[pl.ds(b * CHW, CHW)], sems[b])
            return carry

        lax.fori_loop(0, SEG // 2, pair, 0)
        for b in range(2):
            pltpu.make_async_copy(
                table_hbm.at[src_v.at[pb].at[0]], rows_v.at[pl.ds(b * CHW, CHW)],
                sems[b]).wait()
        if s < CPT // SEG - 1:
            pltpu.make_async_copy(src_hbm.at[wid, pl.ds(0, SEG)],
                                  src_v.at[1 - pb], sem_i).wait()
    plsc.subcore_barrier()
    pltpu.sync_copy(acc.at[pl.ds(sid * RPT, RPT)],
                    out_hbm.at[cid, pl.ds(sid * RPT, RPT)])


@functools.partial(
    pl.kernel,
    mesh=_mesh,
    out_type=jax.ShapeDtypeStruct((2, NP, D16), jnp.float32),
    scratch_types=[
        pltpu.VMEM((CPT, CHW), jnp.int32),
        pltpu.VMEM((CHW, D16), jnp.float32),
        pltpu.VMEM_SHARED((NP, D16), jnp.float32),
        pltpu.SemaphoreType.DMA,
    ],
)
def _degcount(ones_hbm, dst_hbm, zeros16_hbm, out_hbm,
              dst_v, ones_v, acc, sem):
    cid = lax.axis_index("c")
    sid = lax.axis_index("s")
    wid = sid * 2 + cid
    pltpu.sync_copy(zeros16_hbm, acc.at[pl.ds(sid * RPT, RPT)])
    pltpu.sync_copy(ones_hbm, ones_v)
    pltpu.sync_copy(dst_hbm.at[wid], dst_v)
    plsc.subcore_barrier()

    def grp(g, carry):
        pltpu.sync_copy(ones_v, acc.at[dst_v.at[g]], add=True)
        return carry

    lax.fori_loop(0, CPT, grp, 0)
    plsc.subcore_barrier()
    pltpu.sync_copy(acc.at[pl.ds(sid * RPT, RPT)],
                    out_hbm.at[cid, pl.ds(sid * RPT, RPT)])


BM = 1000
NB = N // BM
_row2 = lambda ph, j: (j, 0)
_fix2 = lambda ph, j: (0, 0)
_row1 = lambda j: (j, 0)
_fix1 = lambda j: (0, 0)


def _tc_h0w(x, pe, Wx, Wp, be, W0):
    def body(x_ref, pe_ref, wx_ref, wp_ref, be_ref, w0_ref, o_ref):
        h0 = jnp.dot(x_ref[...], wx_ref[...],
                     preferred_element_type=jnp.float32)
        h0 += jnp.dot(pe_ref[...], wp_ref[...],
                      preferred_element_type=jnp.float32)
        h0 += be_ref[...]
        o_ref[...] = jnp.dot(h0, w0_ref[...],
                             preferred_element_type=jnp.float32)

    return pl.pallas_call(
        body,
        grid=(NB,),
        in_specs=[
            pl.BlockSpec((BM, D), _row1),
            pl.BlockSpec((BM, DPE), _row1),
            pl.BlockSpec((D, D), _fix1),
            pl.BlockSpec((DPE, D), _fix1),
            pl.BlockSpec((1, D), _fix1),
            pl.BlockSpec((D, D), _fix1),
        ],
        out_specs=pl.BlockSpec((BM, D), _row1),
        out_shape=jax.ShapeDtypeStruct((N, D), jnp.float32),
    )(x, pe, Wx, Wp, be, W0)


def _tc_scale(hw0, d0, d1):
    def body(hw_ref, d0_ref, d1_ref, o_ref, dinv_ref):
        dinv = lax.rsqrt(d0_ref[...] + d1_ref[...] + 1.0)
        dinv_ref[...] = dinv
        o_ref[...] = dinv * hw_ref[...]

    return pl.pallas_call(
        body,
        grid=(NB,),
        in_specs=[
            pl.BlockSpec((BM, D), _row1),
            pl.BlockSpec((BM, 1), _row1),
            pl.BlockSpec((BM, 1), _row1),
        ],
        out_specs=[
            pl.BlockSpec((BM, D), _row1),
            pl.BlockSpec((BM, 1), _row1),
        ],
        out_shape=[
            jax.ShapeDtypeStruct((N, D), jnp.float32),
            jax.ShapeDtypeStruct((N, 1), jnp.float32),
        ],
    )(hw0, d0, d1)


def _tc_combine(a0, a1, hs, dinv, b, g, be, Wn):
    def body(a0_ref, a1_ref, hs_ref, dinv_ref, b_ref, g_ref, be_ref, wn_ref,
             o_ref, t_buf, ssum, ssq):
        ph = pl.program_id(0)
        j = pl.program_id(1)

        @pl.when(ph == 0)
        def _():
            t = dinv_ref[...] * (a0_ref[...] + a1_ref[...] + hs_ref[...]) \
                + b_ref[...]
            t_buf[pl.ds(j * BM, BM), :] = t

            @pl.when(j == 0)
            def _():
                ssum[...] = jnp.zeros_like(ssum)
                ssq[...] = jnp.zeros_like(ssq)
            ssum[...] += jnp.sum(t, axis=0, keepdims=True)
            ssq[...] += jnp.sum(t * t, axis=0, keepdims=True)

        @pl.when(ph == 1)
        def _():
            mu = ssum[...] / N
            var = ssq[...] / N - mu * mu
            t = t_buf[pl.ds(j * BM, BM), :]
            hn = jnp.maximum(
                (t - mu) * lax.rsqrt(var + 1e-5) * g_ref[...] + be_ref[...],
                0.0)
            o_ref[...] = dinv_ref[...] * jnp.dot(
                hn, wn_ref[...], preferred_element_type=jnp.float32)

    return pl.pallas_call(
        body,
        grid=(2, NB),
        in_specs=[
            pl.BlockSpec((BM, D), _row2),
            pl.BlockSpec((BM, D), _row2),
            pl.BlockSpec((BM, D), _row2),
            pl.BlockSpec((BM, 1), _row2),
            pl.BlockSpec((1, D), _fix2),
            pl.BlockSpec((1, D), _fix2),
            pl.BlockSpec((1, D), _fix2),
            pl.BlockSpec((D, D), _fix2),
        ],
        out_specs=pl.BlockSpec((BM, D), _row2),
        out_shape=jax.ShapeDtypeStruct((N, D), jnp.float32),
        scratch_shapes=[
            pltpu.VMEM((N, D), jnp.float32),
            pltpu.VMEM((1, D), jnp.float32),
            pltpu.VMEM((1, D), jnp.float32),
        ],
    )(a0, a1, hs, dinv, b, g, be, Wn)


def _tc_final(a0, a1, hs, dinv, b2, Wd, bd):
    def body(a0_ref, a1_ref, hs_ref, dinv_ref, b2_ref, wd_ref, bd_ref, o_ref):
        t = dinv_ref[...] * (a0_ref[...] + a1_ref[...] + hs_ref[...]) \
            + b2_ref[...]
        o_ref[...] = jnp.dot(t, wd_ref[...],
                             preferred_element_type=jnp.float32) + bd_ref[...]

    return pl.pallas_call(
        body,
        grid=(NB,),
        in_specs=[
            pl.BlockSpec((BM, D), _row1),
            pl.BlockSpec((BM, D), _row1),
            pl.BlockSpec((BM, D), _row1),
            pl.BlockSpec((BM, 1), _row1),
            pl.BlockSpec((1, D), _fix1),
            pl.BlockSpec((D, D), _fix1),
            pl.BlockSpec((1, D), _fix1),
        ],
        out_specs=pl.BlockSpec((BM, D), _row1),
        out_shape=jax.ShapeDtypeStruct((N, D), jnp.float32),
    )(a0, a1, hs, dinv, b2, Wd, bd)


def kernel(x, pe, edge_index, W_enc, b_enc, W0, b0, g0, be0,
           W1, b1, g1, be1, W2, b2, W_dec, b_dec):
    src = edge_index[0]
    dst = edge_index[1]

    pad = EP - E
    padix = jax.lax.iota(jnp.int32, pad)
    src3 = jnp.concatenate([src, padix % N]).reshape(NTILES, CPT, CHW)
    dstp = jnp.concatenate([dst, N + padix % (NP - N)])
    dst3 = dstp.reshape(NTILES, CPT, CHW)
    dst3f = dstp.reshape(NTILES, CPT * CHW)
    zeros = jnp.zeros((RPT, D), jnp.float32)
    zeros16 = jnp.zeros((RPT, D16), jnp.float32)
    ones16 = jnp.ones((CHW, D16), jnp.float32)

    d2 = _degcount(ones16, dst3, zeros16)
    hw0 = _tc_h0w(x, pe, W_enc[:D], W_enc[D:], b_enc[None], W0)
    hs0, dinv = _tc_scale(hw0, d2[0, :N, :1], d2[1, :N, :1])

    def conv_agg(hs):
        parts = _segsum(hs, src3, dst3f, zeros)
        return parts[0, :N], parts[1, :N]

    a0, a1 = conv_agg(hs0)
    hs1 = _tc_combine(a0, a1, hs0, dinv, b0[None], g0[None], be0[None], W1)
    a0, a1 = conv_agg(hs1)
    hs2 = _tc_combine(a0, a1, hs1, dinv, b1[None], g1[None], be1[None], W2)
    a0, a1 = conv_agg(hs2)
    return _tc_final(a0, a1, hs2, dinv, b2[None], W_dec, b_dec[None])


# speedup vs baseline: 1.2028x; 1.2028x over previous
import functools

import jax
import jax.numpy as jnp
from jax import lax
from jax.experimental import pallas as pl
from jax.experimental.pallas import tpu as pltpu
from jax.experimental.pallas import tpu_sc as plsc

N = 10000
E = 320000
D = 128
DPE = 4

NTILES = 32
NSUB = 16
NP = 10112
RPT = NP // NSUB
CHW = 128
NCHUNK = 2560
EP = NCHUNK * CHW
CPT = NCHUNK // NTILES
SEG = 16
D16 = 128

_mesh = plsc.VectorSubcoreMesh(core_axis_name="c", subcore_axis_name="s")


@functools.partial(
    pl.kernel,
    mesh=_mesh,
    out_type=jax.ShapeDtypeStruct((2, NP, D), jnp.float32),
    scratch_types=[
        pltpu.VMEM((2, SEG, CHW), jnp.int32),
        pltpu.VMEM((CPT, CHW), jnp.int32),
        pltpu.VMEM((2, CHW, D), jnp.float32),
        pltpu.VMEM_SHARED((NP, D), jnp.float32),
        pltpu.SemaphoreType.DMA,
        pltpu.SemaphoreType.DMA,
        pltpu.SemaphoreType.DMA,
        pltpu.SemaphoreType.DMA,
        pltpu.SemaphoreType.DMA,
    ],
)
def _segsum(table_hbm, src_hbm, dst_hbm, zeros_hbm, out_hbm,
            src_v, dst_v, rows_v, acc, sem0, sem1, sem_s0, sem_s1, sem_i):
    cid = lax.axis_index("c")
    sid = lax.axis_index("s")
    wid = sid * 2 + cid
    pltpu.sync_copy(zeros_hbm, acc.at[pl.ds(sid * RPT, RPT)])
    pltpu.sync_copy(dst_hbm.at[wid], dst_v)
    pltpu.sync_copy(src_hbm.at[wid, pl.ds(0, SEG)], src_v.at[0])
    plsc.subcore_barrier()

    sems = (sem0, sem1)
    ssems = (sem_s0, sem_s1)
    for s in range(CPT // SEG):
        pb = s % 2
        if s < CPT // SEG - 1:
            pltpu.async_copy(src_hbm.at[wid, pl.ds((s + 1) * SEG, SEG)],
                             src_v.at[1 - pb], sem_i)
        for b in range(2):
            pltpu.async_copy(table_hbm.at[src_v.at[pb].at[b]],
                             rows_v.at[b], sems[b])

        def pair(p, carry):
            for b in range(2):
                j = 2 * p + b
                pltpu.make_async_copy(
                    table_hbm.at[src_v.at[pb].at[0]], rows_v.at[b],
                    sems[b]).wait()
                pltpu.sync_copy(rows_v.at[b], acc.at[dst_v.at[s * SEG + j]],
                                add=True)
                nj = jnp.minimum(j + 2, SEG - 1)
                pltpu.async_copy(table_hbm.at[src_v.at[pb].at[nj]],
                                 rows_v.at[b], sems[b])
            return carry

        lax.fori_loop(0, SEG // 2, pair, 0)
        for b in range(2):
            pltpu.make_async_copy(
                table_hbm.at[src_v.at[pb].at[0]], rows_v.at[b],
                sems[b]).wait()
        if s < CPT // SEG - 1:
            pltpu.make_async_copy(src_hbm.at[wid, pl.ds(0, SEG)],
                                  src_v.at[1 - pb], sem_i).wait()
    plsc.subcore_barrier()
    pltpu.sync_copy(acc.at[pl.ds(sid * RPT, RPT)],
                    out_hbm.at[cid, pl.ds(sid * RPT, RPT)])


@functools.partial(
    pl.kernel,
    mesh=_mesh,
    out_type=jax.ShapeDtypeStruct((2, NP, D16), jnp.float32),
    scratch_types=[
        pltpu.VMEM((CPT, CHW), jnp.int32),
        pltpu.VMEM((CHW, D16), jnp.float32),
        pltpu.VMEM_SHARED((NP, D16), jnp.float32),
        pltpu.SemaphoreType.DMA,
    ],
)
def _degcount(ones_hbm, dst_hbm, zeros16_hbm, out_hbm,
              dst_v, ones_v, acc, sem):
    cid = lax.axis_index("c")
    sid = lax.axis_index("s")
    wid = sid * 2 + cid
    pltpu.sync_copy(zeros16_hbm, acc.at[pl.ds(sid * RPT, RPT)])
    pltpu.sync_copy(ones_hbm, ones_v)
    pltpu.sync_copy(dst_hbm.at[wid], dst_v)
    plsc.subcore_barrier()

    def grp(g, carry):
        pltpu.sync_copy(ones_v, acc.at[dst_v.at[g]], add=True)
        return carry

    lax.fori_loop(0, CPT, grp, 0)
    plsc.subcore_barrier()
    pltpu.sync_copy(acc.at[pl.ds(sid * RPT, RPT)],
                    out_hbm.at[cid, pl.ds(sid * RPT, RPT)])


BM = 1000
NB = N // BM
_row2 = lambda ph, j: (j, 0)
_fix2 = lambda ph, j: (0, 0)
_row1 = lambda j: (j, 0)
_fix1 = lambda j: (0, 0)


def _tc_h0w(x, pe, Wx, Wp, be, W0):
    def body(x_ref, pe_ref, wx_ref, wp_ref, be_ref, w0_ref, o_ref):
        h0 = jnp.dot(x_ref[...], wx_ref[...],
                     preferred_element_type=jnp.float32)
        h0 += jnp.dot(pe_ref[...], wp_ref[...],
                      preferred_element_type=jnp.float32)
        h0 += be_ref[...]
        o_ref[...] = jnp.dot(h0, w0_ref[...],
                             preferred_element_type=jnp.float32)

    return pl.pallas_call(
        body,
        grid=(NB,),
        in_specs=[
            pl.BlockSpec((BM, D), _row1),
            pl.BlockSpec((BM, DPE), _row1),
            pl.BlockSpec((D, D), _fix1),
            pl.BlockSpec((DPE, D), _fix1),
            pl.BlockSpec((1, D), _fix1),
            pl.BlockSpec((D, D), _fix1),
        ],
        out_specs=pl.BlockSpec((BM, D), _row1),
        out_shape=jax.ShapeDtypeStruct((N, D), jnp.float32),
    )(x, pe, Wx, Wp, be, W0)


def _tc_scale(hw0, d0, d1):
    def body(hw_ref, d0_ref, d1_ref, o_ref, dinv_ref):
        dinv = lax.rsqrt(d0_ref[...] + d1_ref[...] + 1.0)
        dinv_ref[...] = dinv
        o_ref[...] = dinv * hw_ref[...]

    return pl.pallas_call(
        body,
        grid=(NB,),
        in_specs=[
            pl.BlockSpec((BM, D), _row1),
            pl.BlockSpec((BM, 1), _row1),
            pl.BlockSpec((BM, 1), _row1),
        ],
        out_specs=[
            pl.BlockSpec((BM, D), _row1),
            pl.BlockSpec((BM, 1), _row1),
        ],
        out_shape=[
            jax.ShapeDtypeStruct((N, D), jnp.float32),
            jax.ShapeDtypeStruct((N, 1), jnp.float32),
        ],
    )(hw0, d0, d1)


def _tc_combine(a0, a1, hs, dinv, b, g, be, Wn):
    def body(a0_ref, a1_ref, hs_ref, dinv_ref, b_ref, g_ref, be_ref, wn_ref,
             o_ref, t_buf, ssum, ssq):
        ph = pl.program_id(0)
        j = pl.program_id(1)

        @pl.when(ph == 0)
        def _():
            t = dinv_ref[...] * (a0_ref[...] + a1_ref[...] + hs_ref[...]) \
                + b_ref[...]
            t_buf[pl.ds(j * BM, BM), :] = t

            @pl.when(j == 0)
            def _():
                ssum[...] = jnp.zeros_like(ssum)
                ssq[...] = jnp.zeros_like(ssq)
            ssum[...] += jnp.sum(t, axis=0, keepdims=True)
            ssq[...] += jnp.sum(t * t, axis=0, keepdims=True)

        @pl.when(ph == 1)
        def _():
            mu = ssum[...] / N
            var = ssq[...] / N - mu * mu
            t = t_buf[pl.ds(j * BM, BM), :]
            hn = jnp.maximum(
                (t - mu) * lax.rsqrt(var + 1e-5) * g_ref[...] + be_ref[...],
                0.0)
            o_ref[...] = dinv_ref[...] * jnp.dot(
                hn, wn_ref[...], preferred_element_type=jnp.float32)

    return pl.pallas_call(
        body,
        grid=(2, NB),
        in_specs=[
            pl.BlockSpec((BM, D), _row2),
            pl.BlockSpec((BM, D), _row2),
            pl.BlockSpec((BM, D), _row2),
            pl.BlockSpec((BM, 1), _row2),
            pl.BlockSpec((1, D), _fix2),
            pl.BlockSpec((1, D), _fix2),
            pl.BlockSpec((1, D), _fix2),
            pl.BlockSpec((D, D), _fix2),
        ],
        out_specs=pl.BlockSpec((BM, D), _row2),
        out_shape=jax.ShapeDtypeStruct((N, D), jnp.float32),
        scratch_shapes=[
            pltpu.VMEM((N, D), jnp.float32),
            pltpu.VMEM((1, D), jnp.float32),
            pltpu.VMEM((1, D), jnp.float32),
        ],
    )(a0, a1, hs, dinv, b, g, be, Wn)


def _tc_final(a0, a1, hs, dinv, b2, Wd, bd):
    def body(a0_ref, a1_ref, hs_ref, dinv_ref, b2_ref, wd_ref, bd_ref, o_ref):
        t = dinv_ref[...] * (a0_ref[...] + a1_ref[...] + hs_ref[...]) \
            + b2_ref[...]
        o_ref[...] = jnp.dot(t, wd_ref[...],
                             preferred_element_type=jnp.float32) + bd_ref[...]

    return pl.pallas_call(
        body,
        grid=(NB,),
        in_specs=[
            pl.BlockSpec((BM, D), _row1),
            pl.BlockSpec((BM, D), _row1),
            pl.BlockSpec((BM, D), _row1),
            pl.BlockSpec((BM, 1), _row1),
            pl.BlockSpec((1, D), _fix1),
            pl.BlockSpec((D, D), _fix1),
            pl.BlockSpec((1, D), _fix1),
        ],
        out_specs=pl.BlockSpec((BM, D), _row1),
        out_shape=jax.ShapeDtypeStruct((N, D), jnp.float32),
    )(a0, a1, hs, dinv, b2, Wd, bd)


def kernel(x, pe, edge_index, W_enc, b_enc, W0, b0, g0, be0,
           W1, b1, g1, be1, W2, b2, W_dec, b_dec):
    src = edge_index[0]
    dst = edge_index[1]

    pad = EP - E
    padix = jax.lax.iota(jnp.int32, pad)
    src3 = jnp.concatenate([src, padix % N]).reshape(NTILES, CPT, CHW)
    dst3 = jnp.concatenate([dst, N + padix % (NP - N)]).reshape(
        NTILES, CPT, CHW)
    zeros = jnp.zeros((RPT, D), jnp.float32)
    zeros16 = jnp.zeros((RPT, D16), jnp.float32)
    ones16 = jnp.ones((CHW, D16), jnp.float32)

    d2 = _degcount(ones16, dst3, zeros16)
    hw0 = _tc_h0w(x, pe, W_enc[:D], W_enc[D:], b_enc[None], W0)
    hs0, dinv = _tc_scale(hw0, d2[0, :N, :1], d2[1, :N, :1])

    def conv_agg(hs):
        parts = _segsum(hs, src3, dst3, zeros)
        return parts[0, :N], parts[1, :N]

    a0, a1 = conv_agg(hs0)
    hs1 = _tc_combine(a0, a1, hs0, dinv, b0[None], g0[None], be0[None], W1)
    a0, a1 = conv_agg(hs1)
    hs2 = _tc_combine(a0, a1, hs1, dinv, b1[None], g1[None], be1[None], W2)
    a0, a1 = conv_agg(hs2)
    return _tc_final(a0, a1, hs2, dinv, b2[None], W_dec, b_dec[None])


# speedup vs baseline: 1.2039x; 1.0009x over previous
import functools

import jax
import jax.numpy as jnp
from jax import lax
from jax.experimental import pallas as pl
from jax.experimental.pallas import tpu as pltpu
from jax.experimental.pallas import tpu_sc as plsc

N = 10000
E = 320000
D = 128
DPE = 4

NTILES = 32
NSUB = 16
NP = 10112
RPT = NP // NSUB
CHW = 128
NCHUNK = 2560
EP = NCHUNK * CHW
CPT = NCHUNK // NTILES
SEG = 16
D16 = 128

_mesh = plsc.VectorSubcoreMesh(core_axis_name="c", subcore_axis_name="s")


@functools.partial(
    pl.kernel,
    mesh=_mesh,
    out_type=jax.ShapeDtypeStruct((2, NP, D), jnp.float32),
    scratch_types=[
        pltpu.VMEM((2, SEG, CHW), jnp.int32),
        pltpu.VMEM((CPT, CHW), jnp.int32),
        pltpu.VMEM((2, CHW, D), jnp.float32),
        pltpu.VMEM_SHARED((NP, D), jnp.float32),
        pltpu.SemaphoreType.DMA,
        pltpu.SemaphoreType.DMA,
        pltpu.SemaphoreType.DMA,
        pltpu.SemaphoreType.DMA,
        pltpu.SemaphoreType.DMA,
    ],
)
def _segsum(table_hbm, src_hbm, dst_hbm, zeros_hbm, out_hbm,
            src_v, dst_v, rows_v, acc, sem0, sem1, sem_s0, sem_s1, sem_i):
    cid = lax.axis_index("c")
    sid = lax.axis_index("s")
    wid = sid * 2 + cid
    pltpu.sync_copy(zeros_hbm, acc.at[pl.ds(sid * RPT, RPT)])
    pltpu.sync_copy(dst_hbm.at[wid], dst_v)
    pltpu.sync_copy(src_hbm.at[wid, pl.ds(0, SEG)], src_v.at[0])
    plsc.subcore_barrier()

    sems = (sem0, sem1)
    ssems = (sem_s0, sem_s1)
    for s in range(CPT // SEG):
        pb = s % 2
        if s < CPT // SEG - 1:
            pltpu.async_copy(src_hbm.at[wid, pl.ds((s + 1) * SEG, SEG)],
                             src_v.at[1 - pb], sem_i)
        for b in range(2):
            pltpu.async_copy(table_hbm.at[src_v.at[pb].at[b]],
                             rows_v.at[b], sems[b])

        def pair(p, carry):
            for b in range(2):
                j = 2 * p + b
                pltpu.make_async_copy(
                    table_hbm.at[src_v.at[pb].at[0]], rows_v.at[b],
                    sems[b]).wait()
                pltpu.sync_copy(rows_v.at[b], acc.at[dst_v.at[s * SEG + j]],
                                add=True)
                nj = jnp.minimum(j + 2, SEG - 1)
                pltpu.async_copy(table_hbm.at[src_v.at[pb].at[nj]],
                                 rows_v.at[b], sems[b])
            return carry

        lax.fori_loop(0, SEG // 2, pair, 0)
        for b in range(2):
            pltpu.make_async_copy(
                table_hbm.at[src_v.at[pb].at[0]], rows_v.at[b],
                sems[b]).wait()
        if s < CPT // SEG - 1:
            pltpu.make_async_copy(src_hbm.at[wid, pl.ds(0, SEG)],
                                  src_v.at[1 - pb], sem_i).wait()
    plsc.subcore_barrier()
    pltpu.sync_copy(acc.at[pl.ds(sid * RPT, RPT)],
                    out_hbm.at[cid, pl.ds(sid * RPT, RPT)])


@functools.partial(
    pl.kernel,
    mesh=_mesh,
    out_type=jax.ShapeDtypeStruct((2, NP, D16), jnp.float32),
    scratch_types=[
        pltpu.VMEM((CPT, CHW), jnp.int32),
        pltpu.VMEM((CHW, D16), jnp.float32),
        pltpu.VMEM_SHARED((NP, D16), jnp.float32),
        pltpu.SemaphoreType.DMA,
    ],
)
def _degcount(ones_hbm, dst_hbm, zeros16_hbm, out_hbm,
              dst_v, ones_v, acc, sem):
    cid = lax.axis_index("c")
    sid = lax.axis_index("s")
    wid = sid * 2 + cid
    pltpu.sync_copy(zeros16_hbm, acc.at[pl.ds(sid * RPT, RPT)])
    pltpu.sync_copy(ones_hbm, ones_v)
    pltpu.sync_copy(dst_hbm.at[wid], dst_v)
    plsc.subcore_barrier()

    KD = 8

    def grp(g, carry):
        for b in range(KD):
            pltpu.async_copy(ones_v, acc.at[dst_v.at[g * KD + b]], sem,
                             add=True)
        for b in range(KD):
            pltpu.make_async_copy(ones_v, acc.at[dst_v.at[0]], sem).wait()
        return carry

    lax.fori_loop(0, CPT // KD, grp, 0)
    plsc.subcore_barrier()
    pltpu.sync_copy(acc.at[pl.ds(sid * RPT, RPT)],
                    out_hbm.at[cid, pl.ds(sid * RPT, RPT)])


BM = 1000
NB = N // BM
_row2 = lambda ph, j: (j, 0)
_fix2 = lambda ph, j: (0, 0)
_row1 = lambda j: (j, 0)
_fix1 = lambda j: (0, 0)


def _tc_h0w(x, pe, Wx, Wp, be, W0):
    def body(x_ref, pe_ref, wx_ref, wp_ref, be_ref, w0_ref, o_ref):
        h0 = jnp.dot(x_ref[...], wx_ref[...],
                     preferred_element_type=jnp.float32)
        h0 += jnp.dot(pe_ref[...], wp_ref[...],
                      preferred_element_type=jnp.float32)
        h0 += be_ref[...]
        o_ref[...] = jnp.dot(h0, w0_ref[...],
                             preferred_element_type=jnp.float32)

    return pl.pallas_call(
        body,
        grid=(NB,),
        in_specs=[
            pl.BlockSpec((BM, D), _row1),
            pl.BlockSpec((BM, DPE), _row1),
            pl.BlockSpec((D, D), _fix1),
            pl.BlockSpec((DPE, D), _fix1),
            pl.BlockSpec((1, D), _fix1),
            pl.BlockSpec((D, D), _fix1),
        ],
        out_specs=pl.BlockSpec((BM, D), _row1),
        out_shape=jax.ShapeDtypeStruct((N, D), jnp.float32),
    )(x, pe, Wx, Wp, be, W0)


def _tc_scale(hw0, d0, d1):
    def body(hw_ref, d0_ref, d1_ref, o_ref, dinv_ref):
        dinv = lax.rsqrt(d0_ref[...] + d1_ref[...] + 1.0)
        dinv_ref[...] = dinv
        o_ref[...] = dinv * hw_ref[...]

    return pl.pallas_call(
        body,
        grid=(NB,),
        in_specs=[
            pl.BlockSpec((BM, D), _row1),
            pl.BlockSpec((BM, 1), _row1),
            pl.BlockSpec((BM, 1), _row1),
        ],
        out_specs=[
            pl.BlockSpec((BM, D), _row1),
            pl.BlockSpec((BM, 1), _row1),
        ],
        out_shape=[
            jax.ShapeDtypeStruct((N, D), jnp.float32),
            jax.ShapeDtypeStruct((N, 1), jnp.float32),
        ],
    )(hw0, d0, d1)


def _tc_combine(a0, a1, hs, dinv, b, g, be, Wn):
    def body(a0_ref, a1_ref, hs_ref, dinv_ref, b_ref, g_ref, be_ref, wn_ref,
             o_ref, t_buf, ssum, ssq):
        ph = pl.program_id(0)
        j = pl.program_id(1)

        @pl.when(ph == 0)
        def _():
            t = dinv_ref[...] * (a0_ref[...] + a1_ref[...] + hs_ref[...]) \
                + b_ref[...]
            t_buf[pl.ds(j * BM, BM), :] = t

            @pl.when(j == 0)
            def _():
                ssum[...] = jnp.zeros_like(ssum)
                ssq[...] = jnp.zeros_like(ssq)
            ssum[...] += jnp.sum(t, axis=0, keepdims=True)
            ssq[...] += jnp.sum(t * t, axis=0, keepdims=True)

        @pl.when(ph == 1)
        def _():
            mu = ssum[...] / N
            var = ssq[...] / N - mu * mu
            t = t_buf[pl.ds(j * BM, BM), :]
            hn = jnp.maximum(
                (t - mu) * lax.rsqrt(var + 1e-5) * g_ref[...] + be_ref[...],
                0.0)
            o_ref[...] = dinv_ref[...] * jnp.dot(
                hn, wn_ref[...], preferred_element_type=jnp.float32)

    return pl.pallas_call(
        body,
        grid=(2, NB),
        in_specs=[
            pl.BlockSpec((BM, D), _row2),
            pl.BlockSpec((BM, D), _row2),
            pl.BlockSpec((BM, D), _row2),
            pl.BlockSpec((BM, 1), _row2),
            pl.BlockSpec((1, D), _fix2),
            pl.BlockSpec((1, D), _fix2),
            pl.BlockSpec((1, D), _fix2),
            pl.BlockSpec((D, D), _fix2),
        ],
        out_specs=pl.BlockSpec((BM, D), _row2),
        out_shape=jax.ShapeDtypeStruct((N, D), jnp.float32),
        scratch_shapes=[
            pltpu.VMEM((N, D), jnp.float32),
            pltpu.VMEM((1, D), jnp.float32),
            pltpu.VMEM((1, D), jnp.float32),
        ],
    )(a0, a1, hs, dinv, b, g, be, Wn)


def _tc_final(a0, a1, hs, dinv, b2, Wd, bd):
    def body(a0_ref, a1_ref, hs_ref, dinv_ref, b2_ref, wd_ref, bd_ref, o_ref):
        t = dinv_ref[...] * (a0_ref[...] + a1_ref[...] + hs_ref[...]) \
            + b2_ref[...]
        o_ref[...] = jnp.dot(t, wd_ref[...],
                             preferred_element_type=jnp.float32) + bd_ref[...]

    return pl.pallas_call(
        body,
        grid=(NB,),
        in_specs=[
            pl.BlockSpec((BM, D), _row1),
            pl.BlockSpec((BM, D), _row1),
            pl.BlockSpec((BM, D), _row1),
            pl.BlockSpec((BM, 1), _row1),
            pl.BlockSpec((1, D), _fix1),
            pl.BlockSpec((D, D), _fix1),
            pl.BlockSpec((1, D), _fix1),
        ],
        out_specs=pl.BlockSpec((BM, D), _row1),
        out_shape=jax.ShapeDtypeStruct((N, D), jnp.float32),
    )(a0, a1, hs, dinv, b2, Wd, bd)


def kernel(x, pe, edge_index, W_enc, b_enc, W0, b0, g0, be0,
           W1, b1, g1, be1, W2, b2, W_dec, b_dec):
    src = edge_index[0]
    dst = edge_index[1]

    pad = EP - E
    padix = jax.lax.iota(jnp.int32, pad)
    src3 = jnp.concatenate([src, padix % N]).reshape(NTILES, CPT, CHW)
    dst3 = jnp.concatenate([dst, N + padix % (NP - N)]).reshape(
        NTILES, CPT, CHW)
    zeros = jnp.zeros((RPT, D), jnp.float32)
    zeros16 = jnp.zeros((RPT, D16), jnp.float32)
    ones16 = jnp.ones((CHW, D16), jnp.float32)

    d2 = _degcount(ones16, dst3, zeros16)
    hw0 = _tc_h0w(x, pe, W_enc[:D], W_enc[D:], b_enc[None], W0)
    hs0, dinv = _tc_scale(hw0, d2[0, :N, :1], d2[1, :N, :1])

    def conv_agg(hs):
        parts = _segsum(hs, src3, dst3, zeros)
        return parts[0, :N], parts[1, :N]

    a0, a1 = conv_agg(hs0)
    hs1 = _tc_combine(a0, a1, hs0, dinv, b0[None], g0[None], be0[None], W1)
    a0, a1 = conv_agg(hs1)
    hs2 = _tc_combine(a0, a1, hs1, dinv, b1[None], g1[None], be1[None], W2)
    a0, a1 = conv_agg(hs2)
    return _tc_final(a0, a1, hs2, dinv, b2[None], W_dec, b_dec[None])
